# baseline (speedup 1.0000x reference)
import jax
import jax.numpy as jnp
from jax.experimental import pallas as pl

ACT = 18
_BB = 512


def _fold_weights(Wg_conv, bg_conv, Wh1, bh1, Wh2, bh2, Wh_fc):
    f32 = jnp.float32
    W1t = jnp.transpose(Wh1, (2, 3, 1, 0))
    T1 = jnp.zeros((5, 5, 7, 7, 3, 32), f32)
    for i in range(5):
        for j in range(5):
            T1 = T1.at[i, j, i:i + 3, j:j + 3, :, :].set(W1t)
    W1_full = jnp.transpose(T1, (2, 3, 4, 0, 1, 5)).reshape(147, 800)
    Wg_full = jnp.einsum('pq,co->pcqo', jnp.eye(49, dtype=f32),
                         Wg_conv.T).reshape(147, 196)
    bg_full = jnp.tile(bg_conv, 49)[None, :]
    b1_full = jnp.tile(bh1, 25)[None, :]
    W2t = jnp.transpose(Wh2, (2, 3, 1, 0))
    T2 = jnp.zeros((3, 3, 5, 5, 32, 32), f32)
    for a in range(3):
        for b in range(3):
            T2 = T2.at[a, b, a:a + 3, b:b + 3, :, :].set(W2t)
    W2_full = jnp.transpose(T2, (2, 3, 4, 0, 1, 5)).reshape(800, 288)
    b2_full = jnp.tile(bh2, 9)[None, :]
    Wfc_perm = Wh_fc.reshape(32, 9, 64).transpose(1, 0, 2).reshape(288, 64)
    return W1_full, b1_full, Wg_full, bg_full, W2_full, b2_full, Wfc_perm


def _body(xf, ar, gg, ga, W1, b1, Wg, bg, Wgfc, bgfc, W2, b2, Wfc, bfc,
          Wha, bha, Whc, bhc, Wca, bca, Wcc, bcc,
          act_o, logp_o, ent_o, val_o):
    f32 = jnp.float32
    X = xf[...]
    hg = jnp.maximum(jnp.dot(X, Wg[...], preferred_element_type=f32) + bg[...], 0.0)
    gl = jnp.dot(hg, Wgfc[...], preferred_element_type=f32) + bgfc[...]
    a_g = gl + gg[...]
    mg = jnp.max(a_g, axis=1, keepdims=True)
    eg = jnp.exp(a_g - mg)
    p = eg[:, 1:2] / (eg[:, 0:1] + eg[:, 1:2])
    mask = p > 0.5
    h1 = jnp.maximum(jnp.dot(X, W1[...], preferred_element_type=f32) + b1[...], 0.0)
    h2 = jnp.maximum(jnp.dot(h1, W2[...], preferred_element_type=f32) + b2[...], 0.0)
    feat = jnp.maximum(jnp.dot(h2, Wfc[...], preferred_element_type=f32) + bfc[...], 0.0)
    logits_h = jnp.dot(feat, Wha[...], preferred_element_type=f32) + bha[...]
    value_h = jnp.dot(feat, Whc[...], preferred_element_type=f32) + bhc[...]
    arr = ar[...]
    logits_c = arr * Wca[...] + bca[...]
    value_c = arr * Wcc[...] + bcc[...]
    logits = jnp.where(mask, logits_c, logits_h)
    value = jnp.where(mask, value_c, value_h)
    z = ga[...] + logits
    zmax = jnp.max(z, axis=1, keepdims=True)
    idx = jax.lax.broadcasted_iota(jnp.int32, z.shape, 1)
    action = jnp.min(jnp.where(z == zmax, idx, ACT), axis=1, keepdims=True)
    lmax = jnp.max(logits, axis=1, keepdims=True)
    shifted = logits - lmax
    sumexp = jnp.sum(jnp.exp(shifted), axis=1, keepdims=True)
    logsm = shifted - jnp.log(sumexp)
    logp_a = jnp.sum(jnp.where(idx == action, logsm, 0.0), axis=1, keepdims=True)
    logp_g = jnp.where(mask, jnp.log(p + 1e-8), jnp.log(1.0 - p + 1e-8))
    probs = jnp.exp(logsm)
    ent_c = -jnp.sum(probs * logsm, axis=1, keepdims=True)
    ent_g = -(p * jnp.log(p + 1e-8) + (1.0 - p) * jnp.log(1.0 - p + 1e-8))
    act_o[...] = action
    logp_o[...] = logp_a + logp_g
    ent_o[...] = ent_c + ent_g
    val_o[...] = value


def kernel(x, arrow, Wg_conv, bg_conv, Wg_fc, bg_fc, Wc_act, bc_act,
           Wc_crit, bc_crit, Wh1, bh1, Wh2, bh2, Wh_fc, bh_fc,
           Wh_act, bh_act, Wh_crit, bh_crit):
    f32 = jnp.float32
    B = x.shape[0]
    xf = x.reshape(B, 147)
    g_gate = jax.random.gumbel(jax.random.key(42), (B, 2), f32)
    g_act = jax.random.gumbel(jax.random.key(7), (B, ACT), f32)
    W1, b1, Wg, bg, W2, b2, Wfc = _fold_weights(
        Wg_conv, bg_conv, Wh1, bh1, Wh2, bh2, Wh_fc)
    Wgfc = Wg_fc.reshape(4, 49, 2).transpose(1, 0, 2).reshape(196, 2)

    nb = B // _BB
    row = lambda i: (i, 0)
    full = lambda i: (0, 0)

    def wspec(shape):
        return pl.BlockSpec(shape, full)

    out = pl.pallas_call(
        _body,
        grid=(nb,),
        in_specs=[
            pl.BlockSpec((_BB, 147), row),
            pl.BlockSpec((_BB, 1), row),
            pl.BlockSpec((_BB, 2), row),
            pl.BlockSpec((_BB, ACT), row),
            wspec((147, 800)), wspec((1, 800)),
            wspec((147, 196)), wspec((1, 196)),
            wspec((196, 2)), wspec((1, 2)),
            wspec((800, 288)), wspec((1, 288)),
            wspec((288, 64)), wspec((1, 64)),
            wspec((64, ACT)), wspec((1, ACT)),
            wspec((64, 1)), wspec((1, 1)),
            wspec((1, ACT)), wspec((1, ACT)),
            wspec((1, 1)), wspec((1, 1)),
        ],
        out_specs=[
            pl.BlockSpec((_BB, 1), row),
            pl.BlockSpec((_BB, 1), row),
            pl.BlockSpec((_BB, 1), row),
            pl.BlockSpec((_BB, 1), row),
        ],
        out_shape=[
            jax.ShapeDtypeStruct((B, 1), jnp.int32),
            jax.ShapeDtypeStruct((B, 1), f32),
            jax.ShapeDtypeStruct((B, 1), f32),
            jax.ShapeDtypeStruct((B, 1), f32),
        ],
    )(xf, arrow, g_gate, g_act,
      W1, b1, Wg, bg, Wgfc, bg_fc[None, :],
      W2, b2, Wfc, bh_fc[None, :],
      Wh_act, bh_act[None, :], Wh_crit, bh_crit[None, :],
      Wc_act, bc_act[None, :], Wc_crit, bc_crit[None, :])
    action, logp, entropy, value = out
    return (action[:, 0], logp[:, 0], entropy[:, 0], value)


# speedup vs baseline: 1.2475x; 1.2475x over previous
import jax
import jax.numpy as jnp
from jax.experimental import pallas as pl

ACT = 18
_BB = 512


def _fold_weights(Wg_conv, bg_conv, Wh1, bh1, Wh2, bh2, Wh_fc):
    f32 = jnp.float32
    W1t = jnp.transpose(Wh1, (2, 3, 1, 0))
    cols1 = []
    for i in range(5):
        for j in range(5):
            blk = jnp.zeros((7, 7, 3, 32), f32).at[i:i + 3, j:j + 3].set(W1t)
            cols1.append(blk.reshape(147, 32))
    W1_full = jnp.concatenate(cols1, axis=1)
    Wg_full = jnp.einsum('pq,co->pcqo', jnp.eye(49, dtype=f32),
                         Wg_conv.T).reshape(147, 196)
    bg_full = jnp.tile(bg_conv, 49)[None, :]
    b1_full = jnp.tile(bh1, 25)[None, :]
    W2t = jnp.transpose(Wh2, (2, 3, 1, 0))
    cols2 = []
    for a in range(3):
        for b in range(3):
            blk = jnp.zeros((5, 5, 32, 32), f32).at[a:a + 3, b:b + 3].set(W2t)
            cols2.append(blk.reshape(800, 32))
    W2_full = jnp.concatenate(cols2, axis=1)
    b2_full = jnp.tile(bh2, 9)[None, :]
    Wfc_perm = Wh_fc.reshape(32, 9, 64).transpose(1, 0, 2).reshape(288, 64)
    return W1_full, b1_full, Wg_full, bg_full, W2_full, b2_full, Wfc_perm


def _body(xf, ar, gg, ga, W1, b1, Wg, bg, Wgfc, bgfc, W2, b2, Wfc, bfc,
          Wha, bha, Whc, bhc, Wca, bca, Wcc, bcc,
          act_o, logp_o, ent_o, val_o):
    f32 = jnp.float32
    X = xf[...]
    hg = jnp.maximum(jnp.dot(X, Wg[...], preferred_element_type=f32) + bg[...], 0.0)
    gl = jnp.dot(hg, Wgfc[...], preferred_element_type=f32) + bgfc[...]
    a_g = gl + gg[...]
    mg = jnp.max(a_g, axis=1, keepdims=True)
    eg = jnp.exp(a_g - mg)
    p = eg[:, 1:2] / (eg[:, 0:1] + eg[:, 1:2])
    mask = p > 0.5
    h1 = jnp.maximum(jnp.dot(X, W1[...], preferred_element_type=f32) + b1[...], 0.0)
    h2 = jnp.maximum(jnp.dot(h1, W2[...], preferred_element_type=f32) + b2[...], 0.0)
    feat = jnp.maximum(jnp.dot(h2, Wfc[...], preferred_element_type=f32) + bfc[...], 0.0)
    logits_h = jnp.dot(feat, Wha[...], preferred_element_type=f32) + bha[...]
    value_h = jnp.dot(feat, Whc[...], preferred_element_type=f32) + bhc[...]
    arr = ar[...]
    logits_c = arr * Wca[...] + bca[...]
    value_c = arr * Wcc[...] + bcc[...]
    logits = jnp.where(mask, logits_c, logits_h)
    value = jnp.where(mask, value_c, value_h)
    z = ga[...] + logits
    zmax = jnp.max(z, axis=1, keepdims=True)
    idx = jax.lax.broadcasted_iota(jnp.int32, z.shape, 1)
    action = jnp.min(jnp.where(z == zmax, idx, ACT), axis=1, keepdims=True)
    lmax = jnp.max(logits, axis=1, keepdims=True)
    shifted = logits - lmax
    sumexp = jnp.sum(jnp.exp(shifted), axis=1, keepdims=True)
    logsm = shifted - jnp.log(sumexp)
    logp_a = jnp.sum(jnp.where(idx == action, logsm, 0.0), axis=1, keepdims=True)
    logp_g = jnp.where(mask, jnp.log(p + 1e-8), jnp.log(1.0 - p + 1e-8))
    probs = jnp.exp(logsm)
    ent_c = -jnp.sum(probs * logsm, axis=1, keepdims=True)
    ent_g = -(p * jnp.log(p + 1e-8) + (1.0 - p) * jnp.log(1.0 - p + 1e-8))
    act_o[...] = action
    logp_o[...] = logp_a + logp_g
    ent_o[...] = ent_c + ent_g
    val_o[...] = value


def kernel(x, arrow, Wg_conv, bg_conv, Wg_fc, bg_fc, Wc_act, bc_act,
           Wc_crit, bc_crit, Wh1, bh1, Wh2, bh2, Wh_fc, bh_fc,
           Wh_act, bh_act, Wh_crit, bh_crit):
    f32 = jnp.float32
    B = x.shape[0]
    xf = x.reshape(B, 147)
    with jax.ensure_compile_time_eval():
        g_gate = jax.random.gumbel(jax.random.key(42), (B, 2), f32)
        g_act = jax.random.gumbel(jax.random.key(7), (B, ACT), f32)
    W1, b1, Wg, bg, W2, b2, Wfc = _fold_weights(
        Wg_conv, bg_conv, Wh1, bh1, Wh2, bh2, Wh_fc)
    Wgfc = Wg_fc.reshape(4, 49, 2).transpose(1, 0, 2).reshape(196, 2)

    nb = B // _BB
    row = lambda i: (i, 0)
    full = lambda i: (0, 0)

    def wspec(shape):
        return pl.BlockSpec(shape, full)

    out = pl.pallas_call(
        _body,
        grid=(nb,),
        in_specs=[
            pl.BlockSpec((_BB, 147), row),
            pl.BlockSpec((_BB, 1), row),
            pl.BlockSpec((_BB, 2), row),
            pl.BlockSpec((_BB, ACT), row),
            wspec((147, 800)), wspec((1, 800)),
            wspec((147, 196)), wspec((1, 196)),
            wspec((196, 2)), wspec((1, 2)),
            wspec((800, 288)), wspec((1, 288)),
            wspec((288, 64)), wspec((1, 64)),
            wspec((64, ACT)), wspec((1, ACT)),
            wspec((64, 1)), wspec((1, 1)),
            wspec((1, ACT)), wspec((1, ACT)),
            wspec((1, 1)), wspec((1, 1)),
        ],
        out_specs=[
            pl.BlockSpec((_BB, 1), row),
            pl.BlockSpec((_BB, 1), row),
            pl.BlockSpec((_BB, 1), row),
            pl.BlockSpec((_BB, 1), row),
        ],
        out_shape=[
            jax.ShapeDtypeStruct((B, 1), jnp.int32),
            jax.ShapeDtypeStruct((B, 1), f32),
            jax.ShapeDtypeStruct((B, 1), f32),
            jax.ShapeDtypeStruct((B, 1), f32),
        ],
    )(xf, arrow, g_gate, g_act,
      W1, b1, Wg, bg, Wgfc, bg_fc[None, :],
      W2, b2, Wfc, bh_fc[None, :],
      Wh_act, bh_act[None, :], Wh_crit, bh_crit[None, :],
      Wc_act, bc_act[None, :], Wc_crit, bc_crit[None, :])
    action, logp, entropy, value = out
    return (action[:, 0], logp[:, 0], entropy[:, 0], value)
